# Initial kernel scaffold; baseline (speedup 1.0000x reference)
import jax
import jax.numpy as jnp
from jax.experimental import pallas as pl


def kernel(x, edge_index1, edge_attr1, edge_index2, edge_attr2, We1, be1, Wl1, bl1, Wr1, We2, be2, Wl2, bl2, Wr2):
    raise NotImplementedError("write your pallas kernel here")


# speedup vs baseline: 2.3202x; 2.3202x over previous
import jax
import jax.numpy as jnp
from jax import lax
from jax.experimental import pallas as pl
from jax.experimental.pallas import tpu as pltpu
from jax.experimental.pallas import tpu_sc as plsc

NC = 2
NS = 16
NW = NC * NS
CHUNK = 64
D = 128


def _sc_scatter_x(n_pad, n_chunks, feats, sidx, didx, z):
    rows_pt = n_pad // NS

    def body(x_hbm, sidx_hbm, didx_hbm, z_hbm, out_hbm,
             acc, sidx_v, didx_v, rows_v, sem):
        c = lax.axis_index("c")
        s = lax.axis_index("s")
        wid = c * NS + s
        r0 = s * rows_pt
        pltpu.sync_copy(z_hbm, rows_v)

        def zb(i, carry):
            pltpu.sync_copy(rows_v, acc.at[pl.ds(r0 + i * CHUNK, CHUNK)])
            return carry

        lax.fori_loop(0, rows_pt // CHUNK, zb, 0)
        plsc.subcore_barrier()

        def chunk(j, carry):
            pltpu.sync_copy(sidx_hbm.at[wid, j], sidx_v)
            pltpu.sync_copy(didx_hbm.at[wid, j], didx_v)
            pltpu.async_copy(x_hbm.at[sidx_v], rows_v, sem).wait()
            pltpu.sync_copy(rows_v, acc.at[didx_v], add=True)
            return carry

        lax.fori_loop(0, n_chunks, chunk, 0)
        plsc.subcore_barrier()

        def ob(i, carry):
            pltpu.sync_copy(acc.at[pl.ds(r0 + i * CHUNK, CHUNK)], rows_v)
            pltpu.sync_copy(rows_v, out_hbm.at[c, pl.ds(r0 + i * CHUNK, CHUNK)])
            return carry

        lax.fori_loop(0, rows_pt // CHUNK, ob, 0)

    f = pl.kernel(
        body,
        out_type=jax.ShapeDtypeStruct((NC, n_pad, D), jnp.float32),
        mesh=plsc.VectorSubcoreMesh(core_axis_name="c", subcore_axis_name="s"),
        scratch_types=[
            pltpu.VMEM_SHARED((n_pad, D), jnp.float32),
            pltpu.VMEM((CHUNK,), jnp.int32),
            pltpu.VMEM((CHUNK,), jnp.int32),
            pltpu.VMEM((CHUNK, D), jnp.float32),
            pltpu.SemaphoreType.DMA,
        ],
    )
    return f(feats, sidx, didx, z)


def _sc_scatter_aux(n_pad, n_chunks, attr4, didx, tmpl):
    rows_pt = n_pad // NS

    def body(attr_hbm, didx_hbm, tmpl_hbm, out_hbm, acc, didx_v, attr_v, aux_v):
        c = lax.axis_index("c")
        s = lax.axis_index("s")
        wid = c * NS + s
        r0 = s * rows_pt
        pltpu.sync_copy(tmpl_hbm.at[pl.ds(CHUNK, CHUNK)], aux_v)

        def zb(i, carry):
            pltpu.sync_copy(aux_v, acc.at[pl.ds(r0 + i * CHUNK, CHUNK)])
            return carry

        lax.fori_loop(0, rows_pt // CHUNK, zb, 0)
        pltpu.sync_copy(tmpl_hbm.at[pl.ds(0, CHUNK)], aux_v)
        plsc.subcore_barrier()

        def chunk(j, carry):
            pltpu.sync_copy(didx_hbm.at[wid, j], didx_v)
            pltpu.sync_copy(attr_hbm.at[wid, j], attr_v)
            for r in range(8):
                for k in range(8):
                    aux_v[r * 8 + k, 0:16] = attr_v[r, k * 16:(k + 1) * 16]
            pltpu.sync_copy(aux_v, acc.at[didx_v], add=True)
            return carry

        lax.fori_loop(0, n_chunks, chunk, 0)
        plsc.subcore_barrier()

        def ob(i, carry):
            pltpu.sync_copy(acc.at[pl.ds(r0 + i * CHUNK, CHUNK)], aux_v)
            pltpu.sync_copy(aux_v, out_hbm.at[c, pl.ds(r0 + i * CHUNK, CHUNK)])
            return carry

        lax.fori_loop(0, rows_pt // CHUNK, ob, 0)

    f = pl.kernel(
        body,
        out_type=jax.ShapeDtypeStruct((NC, n_pad, D), jnp.float32),
        mesh=plsc.VectorSubcoreMesh(core_axis_name="c", subcore_axis_name="s"),
        scratch_types=[
            pltpu.VMEM_SHARED((n_pad, D), jnp.float32),
            pltpu.VMEM((CHUNK,), jnp.int32),
            pltpu.VMEM((8, D), jnp.float32),
            pltpu.VMEM((CHUNK, D), jnp.float32),
        ],
    )
    return f(attr4, didx, tmpl)


def _tc_dense(accx, accaux, xin, We, be, Wl, bl, Wr, act):
    n_pad, d = xin.shape
    br = 1024
    grid = (n_pad // br,)

    def body(ax0, ax1, au0, au1, xr, we, be_r, wl, bl_r, wr, out):
        aux = au0[...] + au1[...]
        attr = aux[:, 0:16]
        deg = aux[:, 16:17]
        aggr = (ax0[...] + ax1[...]
                + jnp.dot(attr, we[...], preferred_element_type=jnp.float32)
                + deg * be_r[...])
        aggr = aggr / jnp.maximum(deg, 1.0)
        val = (jnp.dot(aggr, wl[...], preferred_element_type=jnp.float32)
               + bl_r[...]
               + jnp.dot(xr[...], wr[...], preferred_element_type=jnp.float32))
        out[...] = act(val)

    rd = pl.BlockSpec((br, d), lambda i: (i, 0))

    def full(a):
        return pl.BlockSpec(a.shape, lambda i: (0,) * a.ndim)

    be2 = be.reshape(1, -1)
    bl2 = bl.reshape(1, -1)
    return pl.pallas_call(
        body,
        grid=grid,
        in_specs=[rd, rd, rd, rd, rd,
                  full(We), full(be2), full(Wl), full(bl2), full(Wr)],
        out_specs=rd,
        out_shape=jax.ShapeDtypeStruct((n_pad, d), jnp.float32),
    )(accx[0], accx[1], accaux[0], accaux[1], xin,
      We, be2, Wl, bl2, Wr)


def kernel(x, edge_index1, edge_attr1, edge_index2, edge_attr2,
           We1, be1, Wl1, bl1, Wr1, We2, be2, Wl2, bl2, Wr2):
    n, d = x.shape
    e = edge_index1.shape[1]
    da = edge_attr1.shape[1]
    n_pad = -(-n // 1024) * 1024
    e_pad = -(-e // (NW * CHUNK)) * (NW * CHUNK)
    n_chunks = e_pad // (NW * CHUNK)
    pad = e_pad - e

    def prep_edges(ei, ea):
        src = jnp.pad(ei[0].astype(jnp.int32), (0, pad)).reshape(NW, n_chunks, CHUNK)
        dst = jnp.pad(ei[1].astype(jnp.int32), (0, pad), constant_values=n)
        dst = dst.reshape(NW, n_chunks, CHUNK)
        attr4 = jnp.pad(ea, ((0, pad), (0, 0))).reshape(NW, n_chunks, 8, 128)
        return src, dst, attr4

    s1, d1, a1 = prep_edges(edge_index1, edge_attr1)
    s2, d2, a2 = prep_edges(edge_index2, edge_attr2)
    x_pad = jnp.pad(x, ((0, n_pad - n), (0, 0)))
    z = jnp.zeros((CHUNK, D), jnp.float32)
    tmpl = jnp.concatenate([
        jnp.tile(jnp.concatenate([jnp.zeros((1, da), jnp.float32),
                                  jnp.ones((1, 1), jnp.float32),
                                  jnp.zeros((1, D - da - 1), jnp.float32)],
                                 axis=1), (CHUNK, 1)),
        jnp.zeros((CHUNK, D), jnp.float32),
    ])

    ax = _sc_scatter_x(n_pad, n_chunks, x_pad, s1, d1, z)
    au = _sc_scatter_aux(n_pad, n_chunks, a1, d1, tmpl)
    h = _tc_dense(ax, au, x_pad, We1, be1, Wl1, bl1, Wr1, jax.nn.relu)
    ax2 = _sc_scatter_x(n_pad, n_chunks, h, s2, d2, z)
    au2 = _sc_scatter_aux(n_pad, n_chunks, a2, d2, tmpl)
    out = _tc_dense(ax2, au2, h, We2, be2, Wl2, bl2, Wr2, jax.nn.sigmoid)
    return out[:n]
